# baseline (speedup 1.0000x reference)
import functools

import jax
import jax.numpy as jnp
from jax import lax
from jax.experimental import pallas as pl
from jax.experimental.pallas import tpu as pltpu
from jax.experimental.pallas import tpu_sc as plsc

BATCH = 16384
EMBED = 32
PACK = 4
GROW = 128
HIDDEN = 256
OUT = 10

NC = 2
NS = 16
NW = NC * NS
BPW = BATCH // NW
LANES = 16


def _gather_body(svc_hbm, loc_hbm, tim_hbm, ts_hbm, tl_hbm, tt_hbm,
                 out_s, out_l, out_t,
                 idx_v, x_v, sem):
    wid = lax.axis_index("s") * NC + lax.axis_index("c")
    base = wid * BPW
    lane = lax.iota(jnp.int32, LANES)
    zero = jnp.zeros((LANES,), jnp.int32)
    tables = ((svc_hbm, ts_hbm, out_s, True),
              (loc_hbm, tl_hbm, out_l, True),
              (tim_hbm, tt_hbm, out_t, False))
    for ih, th, oh, wide in tables:
        pltpu.sync_copy(ih.at[pl.ds(base, BPW)], idx_v)

        def fire_body(g, _):
            iv = idx_v[pl.ds(g * LANES, LANES)]
            for k in range(LANES):
                sc = jnp.sum(jnp.where(lane == k, iv, zero))
                b = g * LANES + k
                if wide:
                    pltpu.async_copy(th.at[sc >> 5, (sc >> 2) & 7],
                                     x_v.at[b], sem)
                else:
                    pltpu.async_copy(th.at[sc >> 3, sc & 7],
                                     x_v.at[b, pl.ds(0, EMBED)], sem)
            return 0

        lax.fori_loop(0, BPW // LANES, fire_body, 0)

        def drain_body(r, _):
            if wide:
                pltpu.make_async_copy(th.at[0, 0], x_v.at[r], sem).wait()
            else:
                pltpu.make_async_copy(th.at[0, 0],
                                      x_v.at[r, pl.ds(0, EMBED)], sem).wait()
            return 0

        lax.fori_loop(0, BPW, drain_body, 0)
        pltpu.sync_copy(x_v, oh.at[pl.ds(base, BPW)])


_sc_gather = functools.partial(
    pl.kernel,
    out_type=[jax.ShapeDtypeStruct((BATCH, GROW), jnp.float32)] * 3,
    mesh=plsc.VectorSubcoreMesh(core_axis_name="c", subcore_axis_name="s"),
    scratch_types=[
        pltpu.VMEM((BPW,), jnp.int32),
        pltpu.VMEM((BPW, GROW), jnp.float32),
        pltpu.SemaphoreType.DMA,
    ],
    compiler_params=pltpu.CompilerParams(needs_layout_passes=False),
)(_gather_body)


TILE = 2048


def _select(g, sub):
    cols = [g[:, s * EMBED:(s + 1) * EMBED] for s in range(PACK)]
    x = cols[PACK - 1]
    for s in range(PACK - 2, -1, -1):
        x = jnp.where(sub == s, cols[s], x)
    return x


def _mlp_body(si, li, ti, gs, gl, gt, w1, b1, w2, b2, w3, b3, out):
    xs = _select(gs[...], si[...] & 3)
    xl = _select(gl[...], li[...] & 3)
    xt = gt[:, :EMBED]
    del ti
    x = jnp.concatenate([xs, xl, xt], axis=-1)
    h = jnp.dot(x, w1[...], preferred_element_type=jnp.float32) + b1[...]
    h = jnp.maximum(h, 0.0)
    h = jnp.dot(h, w2[...], preferred_element_type=jnp.float32) + b2[...]
    h = jnp.maximum(h, 0.0)
    out[...] = jnp.dot(h, w3[...], preferred_element_type=jnp.float32) + b3[...]


def _mlp(si, li, ti, gs, gl, gt, W1, b1, W2, b2, W3, b3):
    grid = BATCH // TILE
    idx_spec = pl.BlockSpec((TILE, 1), lambda i: (i, 0))
    g_spec = pl.BlockSpec((TILE, GROW), lambda i: (i, 0))
    full = lambda a: pl.BlockSpec(a.shape, lambda i: (0,) * a.ndim)
    return pl.pallas_call(
        _mlp_body,
        grid=(grid,),
        in_specs=[idx_spec, idx_spec, idx_spec, g_spec, g_spec, g_spec,
                  full(W1), full(b1), full(W2), full(b2), full(W3), full(b3)],
        out_specs=pl.BlockSpec((TILE, OUT), lambda i: (i, 0)),
        out_shape=jax.ShapeDtypeStruct((BATCH, OUT), jnp.float32),
    )(si, li, ti, gs, gl, gt, W1, b1, W2, b2, W3, b3)


def kernel(service_idx, location_idx, time_idx, T_service, T_location,
           T_time, W1, b1, W2, b2, W3, b3):
    svc = service_idx.astype(jnp.int32)
    loc = location_idx.astype(jnp.int32)
    tim = time_idx.astype(jnp.int32)
    ts = T_service.reshape(-1, 8, GROW)
    tl = T_location.reshape(-1, 8, GROW)
    tt = T_time.reshape(-1, 8, EMBED)
    gs, gl, gt = _sc_gather(svc, loc, tim, ts, tl, tt)
    return _mlp(svc.reshape(-1, 1), loc.reshape(-1, 1), tim.reshape(-1, 1),
                gs, gl, gt, W1,
                b1.reshape(1, HIDDEN), W2, b2.reshape(1, HIDDEN),
                W3, b3.reshape(1, OUT))


# speedup vs baseline: 2.5266x; 2.5266x over previous
import functools

import jax
import jax.numpy as jnp
from jax import lax
from jax.experimental import pallas as pl
from jax.experimental.pallas import tpu as pltpu
from jax.experimental.pallas import tpu_sc as plsc

BATCH = 16384
EMBED = 32
SUBPACK = 8
HIDDEN = 256
OUT = 10

NC = 2
NS = 16
NW = NC * NS
BPW = BATCH // NW
LANES = 16


def _gather_body(svc_hbm, loc_hbm, tim_hbm, ts_hbm, tl_hbm, tt_hbm,
                 out_s, out_l, out_t,
                 idx_v, x_v, sem):
    wid = lax.axis_index("s") * NC + lax.axis_index("c")
    base = wid * BPW
    lane = lax.iota(jnp.int32, LANES)
    zero = jnp.zeros((LANES,), jnp.int32)
    tables = ((svc_hbm, ts_hbm, out_s),
              (loc_hbm, tl_hbm, out_l),
              (tim_hbm, tt_hbm, out_t))
    for ih, th, oh in tables:
        pltpu.sync_copy(ih.at[pl.ds(base, BPW)], idx_v)

        def fire_body(g, _):
            iv = idx_v[pl.ds(g * LANES, LANES)]
            for k in range(LANES):
                sc = jnp.sum(jnp.where(lane == k, iv, zero))
                pltpu.async_copy(th.at[sc >> 3, sc & 7],
                                 x_v.at[g * LANES + k], sem)
            return 0

        lax.fori_loop(0, BPW // LANES, fire_body, 0)

        def drain_body(g, _):
            pltpu.make_async_copy(oh.at[pl.ds(base, LANES)],
                                  x_v.at[pl.ds(g * LANES, LANES)],
                                  sem).wait()
            return 0

        lax.fori_loop(0, BPW // LANES, drain_body, 0)
        pltpu.sync_copy(x_v, oh.at[pl.ds(base, BPW)])


_sc_gather = functools.partial(
    pl.kernel,
    out_type=[jax.ShapeDtypeStruct((BATCH, EMBED), jnp.float32)] * 3,
    mesh=plsc.VectorSubcoreMesh(core_axis_name="c", subcore_axis_name="s"),
    scratch_types=[
        pltpu.VMEM((BPW,), jnp.int32),
        pltpu.VMEM((BPW, EMBED), jnp.float32),
        pltpu.SemaphoreType.DMA,
    ],
    compiler_params=pltpu.CompilerParams(needs_layout_passes=False),
)(_gather_body)


TILE = 8192


def _mlp_body(xs, xl, xt, w1, b1, w2, b2, w3, b3, out):
    x = jnp.concatenate([xs[...], xl[...], xt[...]], axis=-1)
    h = jnp.dot(x, w1[...], preferred_element_type=jnp.float32) + b1[...]
    h = jnp.maximum(h, 0.0)
    h = jnp.dot(h, w2[...], preferred_element_type=jnp.float32) + b2[...]
    h = jnp.maximum(h, 0.0)
    out[...] = jnp.dot(h, w3[...], preferred_element_type=jnp.float32) + b3[...]


def _mlp(xs, xl, xt, W1, b1, W2, b2, W3, b3):
    grid = BATCH // TILE
    emb_spec = pl.BlockSpec((TILE, EMBED), lambda i: (i, 0))
    full = lambda a: pl.BlockSpec(a.shape, lambda i: (0,) * a.ndim)
    return pl.pallas_call(
        _mlp_body,
        grid=(grid,),
        in_specs=[emb_spec, emb_spec, emb_spec,
                  full(W1), full(b1), full(W2), full(b2), full(W3), full(b3)],
        out_specs=pl.BlockSpec((TILE, OUT), lambda i: (i, 0)),
        out_shape=jax.ShapeDtypeStruct((BATCH, OUT), jnp.float32),
    )(xs, xl, xt, W1, b1, W2, b2, W3, b3)


def kernel(service_idx, location_idx, time_idx, T_service, T_location,
           T_time, W1, b1, W2, b2, W3, b3):
    svc = service_idx.astype(jnp.int32)
    loc = location_idx.astype(jnp.int32)
    tim = time_idx.astype(jnp.int32)
    ts8 = T_service.reshape(-1, SUBPACK, EMBED)
    tl8 = T_location.reshape(-1, SUBPACK, EMBED)
    tt8 = T_time.reshape(-1, SUBPACK, EMBED)
    xs, xl, xt = _sc_gather(svc, loc, tim, ts8, tl8, tt8)
    return _mlp(xs, xl, xt, W1,
                b1.reshape(1, HIDDEN), W2, b2.reshape(1, HIDDEN),
                W3, b3.reshape(1, OUT))
